# Initial kernel scaffold; baseline (speedup 1.0000x reference)
import jax
import jax.numpy as jnp
from jax.experimental import pallas as pl


def kernel(x, edge_index, edge_types, ggnn_W, ggnn_b, gru_Wih, gru_Whh, gru_bih, gru_bhh, gat_W, gat_attn_l, gat_attn_r, gat_b, conv_l1_W, conv_l1_b, avg_conv_l1_W, avg_conv_l1_b, conv_l2_W, conv_l2_b, avg_conv_l2_W, avg_conv_l2_b, conv_l1c_W, conv_l1c_b, avg_conv_l1c_W, avg_conv_l1c_b, conv_l2c_W, conv_l2c_b, avg_conv_l2c_W, avg_conv_l2c_b, mlp_h_W, mlp_h_b, mlp_W, mlp_b):
    raise NotImplementedError("write your pallas kernel here")


# speedup vs baseline: 50.0934x; 50.0934x over previous
import functools

import jax
import jax.numpy as jnp
from jax import lax
from jax.experimental import pallas as pl
from jax.experimental.pallas import tpu as pltpu
from jax.experimental.pallas import tpu_sc as plsc

N = 10000
E = 320000
D = 128
ET = 4
H = 3
STEPS = 8
B = 8
NPG = N // B
CD = 2 * D
TD = 3 * D

NC = 2
NS = 16
NW = NC * NS
EPW = E // NW
CK = 80
NCH = EPW // CK
STRIPE = 624
CSTRIPE = 2496

_INTERP = False

F32 = jnp.float32


def _trans_body(h_ref, w_ref, o_ref):
    h = h_ref[...]
    for t in range(ET):
        o_ref[t] = jnp.dot(h, w_ref[t], preferred_element_type=F32)


@functools.lru_cache(maxsize=None)
def _trans_call():
    RB = 1000
    return pl.pallas_call(
        _trans_body,
        grid=(N // RB,),
        in_specs=[pl.BlockSpec((RB, D), lambda i: (i, 0)),
                  pl.BlockSpec((ET, D, D), lambda i: (0, 0, 0))],
        out_specs=pl.BlockSpec((ET, RB, D), lambda i: (0, i, 0)),
        out_shape=jax.ShapeDtypeStruct((ET, N, D), F32),
        interpret=_INTERP,
    )


def _sigm(v):
    return 1.0 / (1.0 + jnp.exp(-v))


def _gru_body(ap_ref, cb_ref, h_ref, wih_ref, whh_ref, bih_ref, bhh_ref,
              gw_ref, hn_ref, tr_ref):
    a = ap_ref[0] + ap_ref[1] + cb_ref[...]
    h = h_ref[...]
    gi = lax.dot_general(a, wih_ref[...], (((1,), (1,)), ((), ())),
                         preferred_element_type=F32) + bih_ref[...]
    gh = lax.dot_general(h, whh_ref[...], (((1,), (1,)), ((), ())),
                         preferred_element_type=F32) + bhh_ref[...]
    r = _sigm(gi[:, 0:D] + gh[:, 0:D])
    z = _sigm(gi[:, D:2 * D] + gh[:, D:2 * D])
    ng = jnp.tanh(gi[:, 2 * D:TD] + r * gh[:, 2 * D:TD])
    hn = (1.0 - z) * ng + z * h
    hn_ref[...] = hn
    for t in range(ET):
        tr_ref[t] = jnp.dot(hn, gw_ref[t], preferred_element_type=F32)


@functools.lru_cache(maxsize=None)
def _gru_call():
    RB = 1000
    return pl.pallas_call(
        _gru_body,
        grid=(N // RB,),
        in_specs=[pl.BlockSpec((NC, RB, D), lambda i: (0, i, 0)),
                  pl.BlockSpec((RB, D), lambda i: (i, 0)),
                  pl.BlockSpec((RB, D), lambda i: (i, 0)),
                  pl.BlockSpec((TD, D), lambda i: (0, 0)),
                  pl.BlockSpec((TD, D), lambda i: (0, 0)),
                  pl.BlockSpec((1, TD), lambda i: (0, 0)),
                  pl.BlockSpec((1, TD), lambda i: (0, 0)),
                  pl.BlockSpec((ET, D, D), lambda i: (0, 0, 0))],
        out_specs=[pl.BlockSpec((RB, D), lambda i: (i, 0)),
                   pl.BlockSpec((ET, RB, D), lambda i: (0, i, 0))],
        out_shape=[jax.ShapeDtypeStruct((N, D), F32),
                   jax.ShapeDtypeStruct((ET, N, D), F32)],
        interpret=_INTERP,
    )


def _biasc_body(c_ref, b_ref, o_ref):
    c = c_ref[0] + c_ref[1]
    o_ref[...] = lax.dot_general(c, b_ref[...], (((1,), (0,)), ((), ())),
                                 preferred_element_type=F32)


@functools.lru_cache(maxsize=None)
def _biasc_call():
    RB = 1000
    return pl.pallas_call(
        _biasc_body,
        grid=(N // RB,),
        in_specs=[pl.BlockSpec((NC, RB, ET), lambda i: (0, i, 0)),
                  pl.BlockSpec((ET, D), lambda i: (0, 0))],
        out_specs=pl.BlockSpec((RB, D), lambda i: (i, 0)),
        out_shape=jax.ShapeDtypeStruct((N, D), F32),
        interpret=_INTERP,
    )


def _gatpre_body(h_ref, gw_ref, al_ref, ar_ref, f_ref, lr_ref):
    h = h_ref[...]
    f = jnp.dot(h, gw_ref[...], preferred_element_type=F32)
    rows = []
    er_rows = []
    for hh in range(H):
        fh = f[:, hh * D:(hh + 1) * D]
        f_ref[hh] = fh
        rows.append(jnp.sum(fh * al_ref[hh:hh + 1, :], axis=1)[:, None])
        er_rows.append(jnp.sum(fh * ar_ref[hh:hh + 1, :], axis=1)[:, None])
    zero = jnp.zeros_like(rows[0])
    lr_ref[...] = jnp.concatenate(rows + er_rows + [zero, zero], axis=1)


@functools.lru_cache(maxsize=None)
def _gatpre_call():
    RB = 1000
    return pl.pallas_call(
        _gatpre_body,
        grid=(N // RB,),
        in_specs=[pl.BlockSpec((RB, D), lambda i: (i, 0)),
                  pl.BlockSpec((D, H * D), lambda i: (0, 0)),
                  pl.BlockSpec((H, D), lambda i: (0, 0)),
                  pl.BlockSpec((H, D), lambda i: (0, 0))],
        out_specs=[pl.BlockSpec((H, RB, D), lambda i: (0, i, 0)),
                   pl.BlockSpec((RB, 8), lambda i: (i, 0))],
        out_shape=[jax.ShapeDtypeStruct((H, N, D), F32),
                   jax.ShapeDtypeStruct((N, 8), F32)],
        interpret=_INTERP,
    )


def _dsel(lo, lm):
    r = lax.broadcasted_iota(jnp.int32, (lo, lm), 0)
    c = lax.broadcasted_iota(jnp.int32, (lo, lm), 1)
    return jnp.where(c == 2 * r, 1.0, 0.0).astype(F32)


def _conv3(x, wt_ref, b_ref):
    l = x.shape[0]
    y = (jnp.dot(x[0:l - 2], wt_ref[0], preferred_element_type=F32)
         + jnp.dot(x[1:l - 1], wt_ref[1], preferred_element_type=F32)
         + jnp.dot(x[2:l], wt_ref[2], preferred_element_type=F32))
    return jnp.maximum(y + b_ref[...], 0.0)


def _pool3(z, is_max):
    l = z.shape[0]
    if is_max:
        m = jnp.maximum(jnp.maximum(z[0:l - 2], z[1:l - 1]), z[2:l])
    else:
        m = (z[0:l - 2] + z[1:l - 1] + z[2:l]) * (1.0 / 3.0)
    lo = (l - 3) // 2 + 1
    return jnp.dot(_dsel(lo, l - 2), m, preferred_element_type=F32)


def _pool2(z, is_max):
    l = z.shape[0]
    if is_max:
        m = jnp.maximum(z[0:l - 1], z[1:l])
    else:
        m = (z[0:l - 1] + z[1:l]) * 0.5
    lo = (l - 2) // 2 + 1
    return jnp.dot(_dsel(lo, l - 1), m, preferred_element_type=F32)


def _head(x, w1t_ref, b1_ref, w2t_ref, b2_ref, is_max):
    y1 = _pool3(_conv3(x, w1t_ref, b1_ref), is_max)
    y2 = jnp.maximum(jnp.dot(y1, w2t_ref[0], preferred_element_type=F32)
                     + b2_ref[...], 0.0)
    return _pool2(y2, is_max)


def _readout_body(acc_ref, den_ref, x_ref, gb_ref,
                  w1h_ref, b1h_ref, w1ha_ref, b1ha_ref,
                  w2h_ref, b2h_ref, w2ha_ref, b2ha_ref,
                  w1c_ref, b1c_ref, w1ca_ref, b1ca_ref,
                  w2c_ref, b2c_ref, w2ca_ref, b2ca_ref,
                  mhw_ref, mhb_ref, mw_ref, mb_ref, o_ref):
    hi = jnp.zeros((NPG, D), F32)
    for hh in range(H):
        num = acc_ref[0, hh, 0] + acc_ref[1, hh, 0]
        dd = den_ref[0, 0, hh] + den_ref[0, 1, hh]
        hi = hi + num / (dd[:, None] + 1e-9)
    gbsum = jnp.sum(gb_ref[...], axis=0, keepdims=True)
    hi = (hi + gbsum) * (1.0 / 3.0)
    x = x_ref[0]
    c = jnp.concatenate([hi, x], axis=1)

    ymax = _head(hi, w1h_ref, b1h_ref, w2h_ref, b2h_ref, True)
    pavg = _head(hi, w1ha_ref, b1ha_ref, w2ha_ref, b2ha_ref, False)
    zmax = _head(c, w1c_ref, b1c_ref, w2c_ref, b2c_ref, True)
    uavg = _head(c, w1ca_ref, b1ca_ref, w2ca_ref, b2ca_ref, False)

    v1 = jnp.dot(ymax, mhw_ref[...], preferred_element_type=F32) + mhb_ref[...]
    z1 = jnp.dot(zmax, mw_ref[...], preferred_element_type=F32) + mb_ref[...]
    v2 = jnp.dot(pavg, mhw_ref[...], preferred_element_type=F32) + mhb_ref[...]
    z2 = jnp.dot(uavg, mw_ref[...], preferred_element_type=F32) + mb_ref[...]
    b1s = jnp.mean(v1 * z1)
    b2s = jnp.mean(v2 * z2)
    out = _sigm((b1s + b2s) * 0.5)
    o_ref[...] = jnp.full((1, 1, 128), out, F32)


@functools.lru_cache(maxsize=None)
def _readout_call():
    full = lambda *shape: pl.BlockSpec(shape, lambda b: tuple(0 for _ in shape))
    return pl.pallas_call(
        _readout_body,
        grid=(B,),
        in_specs=[pl.BlockSpec((NC, H, 1, NPG, D), lambda b: (0, 0, b, 0, 0)),
                  pl.BlockSpec((1, NC, H, NPG), lambda b: (b, 0, 0, 0)),
                  pl.BlockSpec((1, NPG, D), lambda b: (b, 0, 0)),
                  full(H, D),
                  full(3, D, D), full(1, D), full(3, D, D), full(1, D),
                  full(1, D, D), full(1, D), full(1, D, D), full(1, D),
                  full(3, CD, CD), full(1, CD), full(3, CD, CD), full(1, CD),
                  full(1, CD, CD), full(1, CD), full(1, CD, CD), full(1, CD),
                  full(D, 1), full(1, 1), full(CD, 1), full(1, 1)],
        out_specs=pl.BlockSpec((1, 1, 128), lambda b: (b, 0, 0)),
        out_shape=jax.ShapeDtypeStruct((B, 1, 128), F32),
        interpret=_INTERP,
    )


def _eidx_body(src_ref, et_ref, dst_ref, gix_ref, cix_ref):
    s = src_ref[...]
    t = et_ref[...]
    d = dst_ref[...]
    gix_ref[...] = t * N + s
    cix_ref[...] = d * ET + t


@functools.lru_cache(maxsize=None)
def _eidx_call():
    RE = 2500
    return pl.pallas_call(
        _eidx_body,
        grid=(1,),
        in_specs=[pl.BlockSpec((RE, 128), lambda i: (0, 0))] * 3,
        out_specs=[pl.BlockSpec((RE, 128), lambda i: (0, 0))] * 2,
        out_shape=[jax.ShapeDtypeStruct((RE, 128), jnp.int32)] * 2,
        interpret=_INTERP,
    )


def _mesh():
    return plsc.VectorSubcoreMesh(core_axis_name="c", subcore_axis_name="s")


def _wid_base():
    cid = lax.axis_index("c")
    sid = lax.axis_index("s")
    wid = sid * NC + cid
    return cid, sid, wid * EPW


def _cnt_body(cix_hbm, out_hbm, cixf_v, cidx_v, ones_v, zc_v,
              acc_sh):
    cid, sid, base = _wid_base()
    pltpu.sync_copy(cix_hbm.at[pl.ds(base, EPW)], cixf_v)

    def zfill(i, carry):
        zc_v[pl.ds(i * 16, 16)] = jnp.zeros((16,), F32)
        return carry

    lax.fori_loop(0, 624 // 16, zfill, 0)
    for k in range(CSTRIPE // 624):
        pltpu.sync_copy(zc_v, acc_sh.at[pl.ds(sid * CSTRIPE + k * 624, 624)])

    @pl.when(sid == NS - 1)
    def _():
        pltpu.sync_copy(zc_v.at[pl.ds(0, 64)],
                        acc_sh.at[pl.ds(NS * CSTRIPE, 64)])

    for j in range(CK // 16):
        ones_v[pl.ds(j * 16, 16)] = jnp.ones((16,), F32)
    plsc.subcore_barrier()

    def chunk(g, carry):
        for j in range(CK // 16):
            o = pl.ds(j * 16, 16)
            go = pl.ds(g * CK + j * 16, 16)
            cidx_v[o] = cixf_v[go]
        pltpu.sync_copy(ones_v, acc_sh.at[cidx_v], add=True)
        return carry

    lax.fori_loop(0, NCH, chunk, 0)
    plsc.subcore_barrier()
    obase = cid * (ET * N)
    for k in range(CSTRIPE // 624):
        pltpu.sync_copy(acc_sh.at[pl.ds(sid * CSTRIPE + k * 624, 624)], zc_v)
        pltpu.sync_copy(zc_v,
                        out_hbm.at[pl.ds(obase + sid * CSTRIPE + k * 624,
                                         624)])

    @pl.when(sid == NS - 1)
    def _():
        pltpu.sync_copy(acc_sh.at[pl.ds(NS * CSTRIPE, 64)],
                        zc_v.at[pl.ds(0, 64)])
        pltpu.sync_copy(zc_v.at[pl.ds(0, 64)],
                        out_hbm.at[pl.ds(obase + NS * CSTRIPE, 64)])


@functools.lru_cache(maxsize=None)
def _cnt_call():
    return pl.kernel(
        _cnt_body,
        out_type=jax.ShapeDtypeStruct((NC * ET * N,), F32),
        mesh=_mesh(),
        scratch_types=[pltpu.VMEM((EPW,), jnp.int32),
                       pltpu.VMEM((CK,), jnp.int32),
                       pltpu.VMEM((CK,), F32),
                       pltpu.VMEM((624,), F32),
                       pltpu.VMEM_SHARED((ET * N,), F32)],
        interpret=_INTERP,
    )


def _zero_rows(rows):
    def zrow(i, carry):
        for j in range(D // 16):
            rows[i, pl.ds(j * 16, 16)] = jnp.zeros((16,), F32)
        return carry

    lax.fori_loop(0, CK, zrow, 0)


def _zero_stripe2d(rows, acc_sh, sid):
    for k in range(7):
        pltpu.sync_copy(rows, acc_sh.at[pl.ds(sid * STRIPE + k * CK, CK)])
    pltpu.sync_copy(rows.at[pl.ds(0, 64)],
                    acc_sh.at[pl.ds(sid * STRIPE + 7 * CK, 64)])

    @pl.when(sid == NS - 1)
    def _():
        pltpu.sync_copy(rows.at[pl.ds(0, 16)],
                        acc_sh.at[pl.ds(NS * STRIPE, 16)])


def _agg_body(tab_hbm, gixh_hbm, dst_hbm, out_hbm,
              gixf_v, dst_v, gix0, gix1, db0, db1, rows0, rows1,
              acc_sh, sem0, sem1):
    cid, sid, base = _wid_base()
    pltpu.sync_copy(gixh_hbm.at[pl.ds(base, EPW)], gixf_v)
    pltpu.sync_copy(dst_hbm.at[pl.ds(base, EPW)], dst_v)
    _zero_rows(rows0)
    _zero_stripe2d(rows0, acc_sh, sid)
    plsc.subcore_barrier()

    def prep(g, gix, db):
        for j in range(CK // 16):
            o = pl.ds(j * 16, 16)
            go = pl.ds(g * CK + j * 16, 16)
            gix[o] = gixf_v[go]
            db[o] = dst_v[go]

    def gstart(gix, rows, sem):
        pltpu.make_async_copy(tab_hbm.at[gix], rows, sem).start()

    def gwait(gix, rows, sem):
        pltpu.make_async_copy(tab_hbm.at[gix], rows, sem).wait()

    prep(0, gix0, db0)
    gstart(gix0, rows0, sem0)

    def pair(k, carry):
        prep(2 * k + 1, gix1, db1)
        gstart(gix1, rows1, sem1)
        gwait(gix0, rows0, sem0)
        pltpu.sync_copy(rows0, acc_sh.at[db0], add=True)
        prep(2 * k + 2, gix0, db0)
        gstart(gix0, rows0, sem0)
        gwait(gix1, rows1, sem1)
        pltpu.sync_copy(rows1, acc_sh.at[db1], add=True)
        return carry

    lax.fori_loop(0, (NCH - 1) // 2, pair, 0)
    gwait(gix0, rows0, sem0)
    pltpu.sync_copy(rows0, acc_sh.at[db0], add=True)

    plsc.subcore_barrier()
    for k in range(7):
        o = pl.ds(sid * STRIPE + k * CK, CK)
        pltpu.sync_copy(acc_sh.at[o], rows0)
        pltpu.sync_copy(rows0, out_hbm.at[cid, o])
    ot = pl.ds(sid * STRIPE + 7 * CK, 64)
    pltpu.sync_copy(acc_sh.at[ot], rows0.at[pl.ds(0, 64)])
    pltpu.sync_copy(rows0.at[pl.ds(0, 64)], out_hbm.at[cid, ot])

    @pl.when(sid == NS - 1)
    def _():
        oz = pl.ds(NS * STRIPE, 16)
        pltpu.sync_copy(acc_sh.at[oz], rows1.at[pl.ds(0, 16)])
        pltpu.sync_copy(rows1.at[pl.ds(0, 16)], out_hbm.at[cid, oz])


@functools.lru_cache(maxsize=None)
def _agg_call():
    return pl.kernel(
        _agg_body,
        out_type=jax.ShapeDtypeStruct((NC, N, D), F32),
        mesh=_mesh(),
        scratch_types=[pltpu.VMEM((EPW,), jnp.int32),
                       pltpu.VMEM((EPW,), jnp.int32),
                       pltpu.VMEM((CK,), jnp.int32),
                       pltpu.VMEM((CK,), jnp.int32),
                       pltpu.VMEM((CK,), jnp.int32),
                       pltpu.VMEM((CK,), jnp.int32),
                       pltpu.VMEM((CK, D), F32),
                       pltpu.VMEM((CK, D), F32),
                       pltpu.VMEM_SHARED((N, D), F32),
                       pltpu.SemaphoreType.DMA,
                       pltpu.SemaphoreType.DMA],
        interpret=_INTERP,
    )


def _gat_body(feat_hbm, lr_hbm, src_hbm, dst_hbm,
              acc_out, den_out,
              src_v, dst_v, fix0, fix1, db0, db1, eri0, eri1,
              elg0, elg1, erg0, erg1, ee0, ee1,
              rows0, rows1, acc_sh, den_sh, sem0, sem1):
    cid, sid, base = _wid_base()
    pltpu.sync_copy(src_hbm.at[pl.ds(base, EPW)], src_v)
    pltpu.sync_copy(dst_hbm.at[pl.ds(base, EPW)], dst_v)

    for hh in range(H):
        _zero_rows(rows0)
        _zero_stripe2d(rows0, acc_sh, sid)
        for j in range(CK // 16):
            ee0[pl.ds(j * 16, 16)] = jnp.zeros((16,), F32)
        for k in range(7):
            pltpu.sync_copy(ee0, den_sh.at[pl.ds(sid * STRIPE + k * CK, CK)])
        pltpu.sync_copy(ee0.at[pl.ds(0, 64)],
                        den_sh.at[pl.ds(sid * STRIPE + 7 * CK, 64)])

        @pl.when(sid == NS - 1)
        def _():
            pltpu.sync_copy(ee0.at[pl.ds(0, 16)],
                            den_sh.at[pl.ds(NS * STRIPE, 16)])

        plsc.subcore_barrier()

        def prep(g, fix, db, eri):
            for j in range(CK // 16):
                o = pl.ds(j * 16, 16)
                go = pl.ds(g * CK + j * 16, 16)
                d16 = dst_v[go]
                fix[o] = src_v[go] + hh * N
                db[o] = d16
                eri[o] = d16 + (H + hh) * N

        def gstart(fix, eri, rows, elg, erg, sem):
            pltpu.make_async_copy(feat_hbm.at[fix], rows, sem).start()
            pltpu.make_async_copy(lr_hbm.at[fix], elg, sem).start()
            pltpu.make_async_copy(lr_hbm.at[eri], erg, sem).start()

        def gwait(fix, eri, rows, elg, erg, sem):
            pltpu.make_async_copy(feat_hbm.at[fix], rows, sem).wait()
            pltpu.make_async_copy(lr_hbm.at[fix], elg, sem).wait()
            pltpu.make_async_copy(lr_hbm.at[eri], erg, sem).wait()

        def scale_scatter(db, elg, erg, ee, rows):
            for j in range(CK // 16):
                o = pl.ds(j * 16, 16)
                e = elg[o] + erg[o]
                e = jnp.where(e > 0, e, 0.2 * e)
                ee[o] = jnp.exp(e)

            def sc16(j, carry):
                eev = ee[pl.ds(j * 16, 16)]
                for l in range(16):
                    ee16 = jnp.full((16,), eev[l], F32)
                    i = j * 16 + l
                    for cb in range(D // 16):
                        o = pl.ds(cb * 16, 16)
                        rows[i, o] = rows[i, o] * ee16
                return carry

            lax.fori_loop(0, CK // 16, sc16, 0)
            pltpu.sync_copy(rows, acc_sh.at[db], add=True)
            pltpu.sync_copy(ee, den_sh.at[db], add=True)

        prep(0, fix0, db0, eri0)
        gstart(fix0, eri0, rows0, elg0, erg0, sem0)

        def pair(k, carry):
            prep(2 * k + 1, fix1, db1, eri1)
            gstart(fix1, eri1, rows1, elg1, erg1, sem1)
            gwait(fix0, eri0, rows0, elg0, erg0, sem0)
            scale_scatter(db0, elg0, erg0, ee0, rows0)
            prep(2 * k + 2, fix0, db0, eri0)
            gstart(fix0, eri0, rows0, elg0, erg0, sem0)
            gwait(fix1, eri1, rows1, elg1, erg1, sem1)
            scale_scatter(db1, elg1, erg1, ee1, rows1)
            return carry

        lax.fori_loop(0, (NCH - 1) // 2, pair, 0)
        gwait(fix0, eri0, rows0, elg0, erg0, sem0)
        scale_scatter(db0, elg0, erg0, ee0, rows0)

        plsc.subcore_barrier()
        dbase = (cid * H + hh) * N
        for k in range(7):
            o = pl.ds(sid * STRIPE + k * CK, CK)
            pltpu.sync_copy(acc_sh.at[o], rows0)
            pltpu.sync_copy(rows0, acc_out.at[cid, hh, o])
            pltpu.sync_copy(den_sh.at[o], ee0)
            pltpu.sync_copy(ee0, den_out.at[pl.ds(dbase + sid * STRIPE
                                                  + k * CK, CK)])
        ot = pl.ds(sid * STRIPE + 7 * CK, 64)
        pltpu.sync_copy(acc_sh.at[ot], rows0.at[pl.ds(0, 64)])
        pltpu.sync_copy(rows0.at[pl.ds(0, 64)], acc_out.at[cid, hh, ot])
        pltpu.sync_copy(den_sh.at[ot], ee0.at[pl.ds(0, 64)])
        pltpu.sync_copy(ee0.at[pl.ds(0, 64)],
                        den_out.at[pl.ds(dbase + sid * STRIPE + 7 * CK, 64)])

        @pl.when(sid == NS - 1)
        def _():
            oz = pl.ds(NS * STRIPE, 16)
            pltpu.sync_copy(acc_sh.at[oz], rows1.at[pl.ds(0, 16)])
            pltpu.sync_copy(rows1.at[pl.ds(0, 16)], acc_out.at[cid, hh, oz])
            pltpu.sync_copy(den_sh.at[oz], ee1.at[pl.ds(0, 16)])
            pltpu.sync_copy(ee1.at[pl.ds(0, 16)],
                            den_out.at[pl.ds(dbase + NS * STRIPE, 16)])
        plsc.subcore_barrier()


@functools.lru_cache(maxsize=None)
def _gat_call():
    return pl.kernel(
        _gat_body,
        out_type=(jax.ShapeDtypeStruct((NC, H, N, D), F32),
                  jax.ShapeDtypeStruct((NC * H * N,), F32)),
        mesh=_mesh(),
        scratch_types=[pltpu.VMEM((EPW,), jnp.int32),
                       pltpu.VMEM((EPW,), jnp.int32),
                       pltpu.VMEM((CK,), jnp.int32),
                       pltpu.VMEM((CK,), jnp.int32),
                       pltpu.VMEM((CK,), jnp.int32),
                       pltpu.VMEM((CK,), jnp.int32),
                       pltpu.VMEM((CK,), jnp.int32),
                       pltpu.VMEM((CK,), jnp.int32),
                       pltpu.VMEM((CK,), F32),
                       pltpu.VMEM((CK,), F32),
                       pltpu.VMEM((CK,), F32),
                       pltpu.VMEM((CK,), F32),
                       pltpu.VMEM((CK,), F32),
                       pltpu.VMEM((CK,), F32),
                       pltpu.VMEM((CK, D), F32),
                       pltpu.VMEM((CK, D), F32),
                       pltpu.VMEM_SHARED((N, D), F32),
                       pltpu.VMEM_SHARED((N,), F32),
                       pltpu.SemaphoreType.DMA,
                       pltpu.SemaphoreType.DMA],
        interpret=_INTERP,
    )


def kernel(x, edge_index, edge_types, ggnn_W, ggnn_b, gru_Wih, gru_Whh,
           gru_bih, gru_bhh, gat_W, gat_attn_l, gat_attn_r, gat_b,
           conv_l1_W, conv_l1_b, avg_conv_l1_W, avg_conv_l1_b,
           conv_l2_W, conv_l2_b, avg_conv_l2_W, avg_conv_l2_b,
           conv_l1c_W, conv_l1c_b, avg_conv_l1c_W, avg_conv_l1c_b,
           conv_l2c_W, conv_l2c_b, avg_conv_l2c_W, avg_conv_l2c_b,
           mlp_h_W, mlp_h_b, mlp_W, mlp_b):
    src = edge_index[0]
    dst = edge_index[1]
    gixm, cixm = _eidx_call()(src.reshape(2500, 128),
                              edge_types.reshape(2500, 128),
                              dst.reshape(2500, 128))
    gix = gixm.reshape(E)
    cix = cixm.reshape(E)

    cntp = _cnt_call()(cix)
    biasc = _biasc_call()(cntp.reshape(NC, N, ET), ggnn_b)

    bih = gru_bih.reshape(1, TD)
    bhh = gru_bhh.reshape(1, TD)

    h = x
    trans = _trans_call()(x, ggnn_W)
    for _step in range(STEPS):
        apart = _agg_call()(trans.reshape(ET * N, D), gix, dst)
        h, trans = _gru_call()(apart, biasc, h, gru_Wih, gru_Whh, bih, bhh,
                               ggnn_W)

    feat, lrt = _gatpre_call()(h, gat_W, gat_attn_l, gat_attn_r)
    lr = lrt.T.reshape(-1)
    acc, den = _gat_call()(feat.reshape(H * N, D), lr, src, dst)

    t210 = lambda w: jnp.transpose(w, (2, 1, 0))
    r1 = lambda v: v.reshape(1, -1)
    out = _readout_call()(
        acc.reshape(NC, H, B, NPG, D),
        den.reshape(NC, H, B, NPG).transpose(2, 0, 1, 3),
        x.reshape(B, NPG, D), gat_b.reshape(H, D),
        t210(conv_l1_W), r1(conv_l1_b), t210(avg_conv_l1_W), r1(avg_conv_l1_b),
        t210(conv_l2_W), r1(conv_l2_b), t210(avg_conv_l2_W), r1(avg_conv_l2_b),
        t210(conv_l1c_W), r1(conv_l1c_b),
        t210(avg_conv_l1c_W), r1(avg_conv_l1c_b),
        t210(conv_l2c_W), r1(conv_l2c_b),
        t210(avg_conv_l2c_W), r1(avg_conv_l2c_b),
        mlp_h_W.reshape(D, 1), mlp_h_b.reshape(1, 1),
        mlp_W.reshape(CD, 1), mlp_b.reshape(1, 1))
    return out[:, 0, 0]


# speedup vs baseline: 50.4938x; 1.0080x over previous
import functools

import jax
import jax.numpy as jnp
from jax import lax
from jax.experimental import pallas as pl
from jax.experimental.pallas import tpu as pltpu
from jax.experimental.pallas import tpu_sc as plsc

N = 10000
E = 320000
D = 128
ET = 4
H = 3
STEPS = 8
B = 8
NPG = N // B
CD = 2 * D
TD = 3 * D

NC = 2
NS = 16
NW = NC * NS
EPW = E // NW
CK = 80
NCH = EPW // CK
STRIPE = 624
CSTRIPE = 2496

_INTERP = False

F32 = jnp.float32


def _trans_body(h_ref, w_ref, b_ref, o_ref):
    h = h_ref[...]
    for t in range(ET):
        o_ref[t] = (jnp.dot(h, w_ref[t], preferred_element_type=F32)
                    + b_ref[t:t + 1, :])


@functools.lru_cache(maxsize=None)
def _trans_call():
    RB = 1000
    return pl.pallas_call(
        _trans_body,
        grid=(N // RB,),
        in_specs=[pl.BlockSpec((RB, D), lambda i: (i, 0)),
                  pl.BlockSpec((ET, D, D), lambda i: (0, 0, 0)),
                  pl.BlockSpec((ET, D), lambda i: (0, 0))],
        out_specs=pl.BlockSpec((ET, RB, D), lambda i: (0, i, 0)),
        out_shape=jax.ShapeDtypeStruct((ET, N, D), F32),
        interpret=_INTERP,
    )


def _sigm(v):
    return 1.0 / (1.0 + jnp.exp(-v))


def _gru_body(ap_ref, h_ref, wih_ref, whh_ref, bih_ref, bhh_ref,
              gw_ref, gb_ref, hn_ref, tr_ref):
    a = ap_ref[0] + ap_ref[1]
    h = h_ref[...]
    gi = lax.dot_general(a, wih_ref[...], (((1,), (1,)), ((), ())),
                         preferred_element_type=F32) + bih_ref[...]
    gh = lax.dot_general(h, whh_ref[...], (((1,), (1,)), ((), ())),
                         preferred_element_type=F32) + bhh_ref[...]
    r = _sigm(gi[:, 0:D] + gh[:, 0:D])
    z = _sigm(gi[:, D:2 * D] + gh[:, D:2 * D])
    ng = jnp.tanh(gi[:, 2 * D:TD] + r * gh[:, 2 * D:TD])
    hn = (1.0 - z) * ng + z * h
    hn_ref[...] = hn
    for t in range(ET):
        tr_ref[t] = (jnp.dot(hn, gw_ref[t], preferred_element_type=F32)
                     + gb_ref[t:t + 1, :])


@functools.lru_cache(maxsize=None)
def _gru_call():
    RB = 1000
    return pl.pallas_call(
        _gru_body,
        grid=(N // RB,),
        in_specs=[pl.BlockSpec((NC, RB, D), lambda i: (0, i, 0)),
                  pl.BlockSpec((RB, D), lambda i: (i, 0)),
                  pl.BlockSpec((TD, D), lambda i: (0, 0)),
                  pl.BlockSpec((TD, D), lambda i: (0, 0)),
                  pl.BlockSpec((1, TD), lambda i: (0, 0)),
                  pl.BlockSpec((1, TD), lambda i: (0, 0)),
                  pl.BlockSpec((ET, D, D), lambda i: (0, 0, 0)),
                  pl.BlockSpec((ET, D), lambda i: (0, 0))],
        out_specs=[pl.BlockSpec((RB, D), lambda i: (i, 0)),
                   pl.BlockSpec((ET, RB, D), lambda i: (0, i, 0))],
        out_shape=[jax.ShapeDtypeStruct((N, D), F32),
                   jax.ShapeDtypeStruct((ET, N, D), F32)],
        interpret=_INTERP,
    )


def _gatpre_body(h_ref, gw_ref, al_ref, ar_ref, f_ref, lr_ref):
    h = h_ref[...]
    f = jnp.dot(h, gw_ref[...], preferred_element_type=F32)
    rows = []
    er_rows = []
    for hh in range(H):
        fh = f[:, hh * D:(hh + 1) * D]
        f_ref[hh] = fh
        rows.append(jnp.sum(fh * al_ref[hh:hh + 1, :], axis=1)[:, None])
        er_rows.append(jnp.sum(fh * ar_ref[hh:hh + 1, :], axis=1)[:, None])
    zero = jnp.zeros_like(rows[0])
    lr_ref[...] = jnp.concatenate(rows + er_rows + [zero, zero], axis=1)


@functools.lru_cache(maxsize=None)
def _gatpre_call():
    RB = 1000
    return pl.pallas_call(
        _gatpre_body,
        grid=(N // RB,),
        in_specs=[pl.BlockSpec((RB, D), lambda i: (i, 0)),
                  pl.BlockSpec((D, H * D), lambda i: (0, 0)),
                  pl.BlockSpec((H, D), lambda i: (0, 0)),
                  pl.BlockSpec((H, D), lambda i: (0, 0))],
        out_specs=[pl.BlockSpec((H, RB, D), lambda i: (0, i, 0)),
                   pl.BlockSpec((RB, 8), lambda i: (i, 0))],
        out_shape=[jax.ShapeDtypeStruct((H, N, D), F32),
                   jax.ShapeDtypeStruct((N, 8), F32)],
        interpret=_INTERP,
    )


def _dsel(lo, lm):
    r = lax.broadcasted_iota(jnp.int32, (lo, lm), 0)
    c = lax.broadcasted_iota(jnp.int32, (lo, lm), 1)
    return jnp.where(c == 2 * r, 1.0, 0.0).astype(F32)


def _conv3(x, wt_ref, b_ref):
    l = x.shape[0]
    y = (jnp.dot(x[0:l - 2], wt_ref[0], preferred_element_type=F32)
         + jnp.dot(x[1:l - 1], wt_ref[1], preferred_element_type=F32)
         + jnp.dot(x[2:l], wt_ref[2], preferred_element_type=F32))
    return jnp.maximum(y + b_ref[...], 0.0)


def _pool3(z, is_max):
    l = z.shape[0]
    if is_max:
        m = jnp.maximum(jnp.maximum(z[0:l - 2], z[1:l - 1]), z[2:l])
    else:
        m = (z[0:l - 2] + z[1:l - 1] + z[2:l]) * (1.0 / 3.0)
    lo = (l - 3) // 2 + 1
    return jnp.dot(_dsel(lo, l - 2), m, preferred_element_type=F32)


def _pool2(z, is_max):
    l = z.shape[0]
    if is_max:
        m = jnp.maximum(z[0:l - 1], z[1:l])
    else:
        m = (z[0:l - 1] + z[1:l]) * 0.5
    lo = (l - 2) // 2 + 1
    return jnp.dot(_dsel(lo, l - 1), m, preferred_element_type=F32)


def _head(x, w1t_ref, b1_ref, w2t_ref, b2_ref, is_max):
    y1 = _pool3(_conv3(x, w1t_ref, b1_ref), is_max)
    y2 = jnp.maximum(jnp.dot(y1, w2t_ref[0], preferred_element_type=F32)
                     + b2_ref[...], 0.0)
    return _pool2(y2, is_max)


def _readout_body(acc_ref, den_ref, x_ref, gb_ref,
                  w1h_ref, b1h_ref, w1ha_ref, b1ha_ref,
                  w2h_ref, b2h_ref, w2ha_ref, b2ha_ref,
                  w1c_ref, b1c_ref, w1ca_ref, b1ca_ref,
                  w2c_ref, b2c_ref, w2ca_ref, b2ca_ref,
                  mhw_ref, mhb_ref, mw_ref, mb_ref, o_ref):
    hi = jnp.zeros((NPG, D), F32)
    for hh in range(H):
        num = acc_ref[0, hh, 0] + acc_ref[1, hh, 0]
        dd = den_ref[0, 0, hh] + den_ref[0, 1, hh]
        hi = hi + num / (dd[:, None] + 1e-9)
    gbsum = jnp.sum(gb_ref[...], axis=0, keepdims=True)
    hi = (hi + gbsum) * (1.0 / 3.0)
    x = x_ref[0]
    c = jnp.concatenate([hi, x], axis=1)

    ymax = _head(hi, w1h_ref, b1h_ref, w2h_ref, b2h_ref, True)
    pavg = _head(hi, w1ha_ref, b1ha_ref, w2ha_ref, b2ha_ref, False)
    zmax = _head(c, w1c_ref, b1c_ref, w2c_ref, b2c_ref, True)
    uavg = _head(c, w1ca_ref, b1ca_ref, w2ca_ref, b2ca_ref, False)

    v1 = jnp.dot(ymax, mhw_ref[...], preferred_element_type=F32) + mhb_ref[...]
    z1 = jnp.dot(zmax, mw_ref[...], preferred_element_type=F32) + mb_ref[...]
    v2 = jnp.dot(pavg, mhw_ref[...], preferred_element_type=F32) + mhb_ref[...]
    z2 = jnp.dot(uavg, mw_ref[...], preferred_element_type=F32) + mb_ref[...]
    b1s = jnp.mean(v1 * z1)
    b2s = jnp.mean(v2 * z2)
    out = _sigm((b1s + b2s) * 0.5)
    o_ref[...] = jnp.full((1, 1, 128), out, F32)


@functools.lru_cache(maxsize=None)
def _readout_call():
    full = lambda *shape: pl.BlockSpec(shape, lambda b: tuple(0 for _ in shape))
    return pl.pallas_call(
        _readout_body,
        grid=(B,),
        in_specs=[pl.BlockSpec((NC, H, 1, NPG, D), lambda b: (0, 0, b, 0, 0)),
                  pl.BlockSpec((1, NC, H, NPG), lambda b: (b, 0, 0, 0)),
                  pl.BlockSpec((1, NPG, D), lambda b: (b, 0, 0)),
                  full(H, D),
                  full(3, D, D), full(1, D), full(3, D, D), full(1, D),
                  full(1, D, D), full(1, D), full(1, D, D), full(1, D),
                  full(3, CD, CD), full(1, CD), full(3, CD, CD), full(1, CD),
                  full(1, CD, CD), full(1, CD), full(1, CD, CD), full(1, CD),
                  full(D, 1), full(1, 1), full(CD, 1), full(1, 1)],
        out_specs=pl.BlockSpec((1, 1, 128), lambda b: (b, 0, 0)),
        out_shape=jax.ShapeDtypeStruct((B, 1, 128), F32),
        interpret=_INTERP,
    )


def _eidx_body(src_ref, et_ref, gix_ref):
    gix_ref[...] = et_ref[...] * N + src_ref[...]


@functools.lru_cache(maxsize=None)
def _eidx_call():
    RE = 2500
    return pl.pallas_call(
        _eidx_body,
        grid=(1,),
        in_specs=[pl.BlockSpec((RE, 128), lambda i: (0, 0))] * 2,
        out_specs=pl.BlockSpec((RE, 128), lambda i: (0, 0)),
        out_shape=jax.ShapeDtypeStruct((RE, 128), jnp.int32),
        interpret=_INTERP,
    )


def _mesh():
    return plsc.VectorSubcoreMesh(core_axis_name="c", subcore_axis_name="s")


def _wid_base():
    cid = lax.axis_index("c")
    sid = lax.axis_index("s")
    wid = sid * NC + cid
    return cid, sid, wid * EPW


def _zero_rows(rows):
    def zrow(i, carry):
        for j in range(D // 16):
            rows[i, pl.ds(j * 16, 16)] = jnp.zeros((16,), F32)
        return carry

    lax.fori_loop(0, CK, zrow, 0)


def _zero_stripe2d(rows, acc_sh, sid):
    for k in range(7):
        pltpu.sync_copy(rows, acc_sh.at[pl.ds(sid * STRIPE + k * CK, CK)])
    pltpu.sync_copy(rows.at[pl.ds(0, 64)],
                    acc_sh.at[pl.ds(sid * STRIPE + 7 * CK, 64)])

    @pl.when(sid == NS - 1)
    def _():
        pltpu.sync_copy(rows.at[pl.ds(0, 16)],
                        acc_sh.at[pl.ds(NS * STRIPE, 16)])


def _agg_body(tab_hbm, gixh_hbm, dst_hbm, out_hbm,
              gixf_v, dst_v, gix0, gix1, db0, db1, rows0, rows1,
              acc_sh, sem0, sem1):
    cid, sid, base = _wid_base()
    pltpu.sync_copy(gixh_hbm.at[pl.ds(base, EPW)], gixf_v)
    pltpu.sync_copy(dst_hbm.at[pl.ds(base, EPW)], dst_v)
    _zero_rows(rows0)
    _zero_stripe2d(rows0, acc_sh, sid)
    plsc.subcore_barrier()

    def prep(g, gix, db):
        for j in range(CK // 16):
            o = pl.ds(j * 16, 16)
            go = pl.ds(g * CK + j * 16, 16)
            gix[o] = gixf_v[go]
            db[o] = dst_v[go]

    def gstart(gix, rows, sem):
        pltpu.make_async_copy(tab_hbm.at[gix], rows, sem).start()

    def gwait(gix, rows, sem):
        pltpu.make_async_copy(tab_hbm.at[gix], rows, sem).wait()

    prep(0, gix0, db0)
    gstart(gix0, rows0, sem0)

    def pair(k, carry):
        prep(2 * k + 1, gix1, db1)
        gstart(gix1, rows1, sem1)
        gwait(gix0, rows0, sem0)
        pltpu.sync_copy(rows0, acc_sh.at[db0], add=True)
        prep(2 * k + 2, gix0, db0)
        gstart(gix0, rows0, sem0)
        gwait(gix1, rows1, sem1)
        pltpu.sync_copy(rows1, acc_sh.at[db1], add=True)
        return carry

    lax.fori_loop(0, (NCH - 1) // 2, pair, 0)
    gwait(gix0, rows0, sem0)
    pltpu.sync_copy(rows0, acc_sh.at[db0], add=True)

    plsc.subcore_barrier()
    for k in range(7):
        o = pl.ds(sid * STRIPE + k * CK, CK)
        pltpu.sync_copy(acc_sh.at[o], rows0)
        pltpu.sync_copy(rows0, out_hbm.at[cid, o])
    ot = pl.ds(sid * STRIPE + 7 * CK, 64)
    pltpu.sync_copy(acc_sh.at[ot], rows0.at[pl.ds(0, 64)])
    pltpu.sync_copy(rows0.at[pl.ds(0, 64)], out_hbm.at[cid, ot])

    @pl.when(sid == NS - 1)
    def _():
        oz = pl.ds(NS * STRIPE, 16)
        pltpu.sync_copy(acc_sh.at[oz], rows1.at[pl.ds(0, 16)])
        pltpu.sync_copy(rows1.at[pl.ds(0, 16)], out_hbm.at[cid, oz])


@functools.lru_cache(maxsize=None)
def _agg_call():
    return pl.kernel(
        _agg_body,
        out_type=jax.ShapeDtypeStruct((NC, N, D), F32),
        mesh=_mesh(),
        scratch_types=[pltpu.VMEM((EPW,), jnp.int32),
                       pltpu.VMEM((EPW,), jnp.int32),
                       pltpu.VMEM((CK,), jnp.int32),
                       pltpu.VMEM((CK,), jnp.int32),
                       pltpu.VMEM((CK,), jnp.int32),
                       pltpu.VMEM((CK,), jnp.int32),
                       pltpu.VMEM((CK, D), F32),
                       pltpu.VMEM((CK, D), F32),
                       pltpu.VMEM_SHARED((N, D), F32),
                       pltpu.SemaphoreType.DMA,
                       pltpu.SemaphoreType.DMA],
        interpret=_INTERP,
    )


def _gat_body(feat_hbm, lr_hbm, src_hbm, dst_hbm,
              acc_out, den_out,
              src_v, dst_v, fix0, fix1, db0, db1, eri0, eri1,
              elg0, elg1, erg0, erg1, ee0, ee1,
              rows0, rows1, acc_sh, den_sh, sem0, sem1):
    cid, sid, base = _wid_base()
    pltpu.sync_copy(src_hbm.at[pl.ds(base, EPW)], src_v)
    pltpu.sync_copy(dst_hbm.at[pl.ds(base, EPW)], dst_v)

    for hh in range(H):
        _zero_rows(rows0)
        _zero_stripe2d(rows0, acc_sh, sid)
        for j in range(CK // 16):
            ee0[pl.ds(j * 16, 16)] = jnp.zeros((16,), F32)
        for k in range(7):
            pltpu.sync_copy(ee0, den_sh.at[pl.ds(sid * STRIPE + k * CK, CK)])
        pltpu.sync_copy(ee0.at[pl.ds(0, 64)],
                        den_sh.at[pl.ds(sid * STRIPE + 7 * CK, 64)])

        @pl.when(sid == NS - 1)
        def _():
            pltpu.sync_copy(ee0.at[pl.ds(0, 16)],
                            den_sh.at[pl.ds(NS * STRIPE, 16)])

        plsc.subcore_barrier()

        def prep(g, fix, db, eri):
            for j in range(CK // 16):
                o = pl.ds(j * 16, 16)
                go = pl.ds(g * CK + j * 16, 16)
                d16 = dst_v[go]
                fix[o] = src_v[go] + hh * N
                db[o] = d16
                eri[o] = d16 + (H + hh) * N

        def gstart(fix, eri, rows, elg, erg, sem):
            pltpu.make_async_copy(feat_hbm.at[fix], rows, sem).start()
            pltpu.make_async_copy(lr_hbm.at[fix], elg, sem).start()
            pltpu.make_async_copy(lr_hbm.at[eri], erg, sem).start()

        def gwait(fix, eri, rows, elg, erg, sem):
            pltpu.make_async_copy(feat_hbm.at[fix], rows, sem).wait()
            pltpu.make_async_copy(lr_hbm.at[fix], elg, sem).wait()
            pltpu.make_async_copy(lr_hbm.at[eri], erg, sem).wait()

        def scale_scatter(db, elg, erg, ee, rows):
            for j in range(CK // 16):
                o = pl.ds(j * 16, 16)
                e = elg[o] + erg[o]
                e = jnp.where(e > 0, e, 0.2 * e)
                ee[o] = jnp.exp(e)

            def sc16(j, carry):
                eev = ee[pl.ds(j * 16, 16)]
                for l in range(16):
                    ee16 = jnp.full((16,), eev[l], F32)
                    i = j * 16 + l
                    for cb in range(D // 16):
                        o = pl.ds(cb * 16, 16)
                        rows[i, o] = rows[i, o] * ee16
                return carry

            lax.fori_loop(0, CK // 16, sc16, 0)
            pltpu.sync_copy(rows, acc_sh.at[db], add=True)
            pltpu.sync_copy(ee, den_sh.at[db], add=True)

        prep(0, fix0, db0, eri0)
        gstart(fix0, eri0, rows0, elg0, erg0, sem0)

        def pair(k, carry):
            prep(2 * k + 1, fix1, db1, eri1)
            gstart(fix1, eri1, rows1, elg1, erg1, sem1)
            gwait(fix0, eri0, rows0, elg0, erg0, sem0)
            scale_scatter(db0, elg0, erg0, ee0, rows0)
            prep(2 * k + 2, fix0, db0, eri0)
            gstart(fix0, eri0, rows0, elg0, erg0, sem0)
            gwait(fix1, eri1, rows1, elg1, erg1, sem1)
            scale_scatter(db1, elg1, erg1, ee1, rows1)
            return carry

        lax.fori_loop(0, (NCH - 1) // 2, pair, 0)
        gwait(fix0, eri0, rows0, elg0, erg0, sem0)
        scale_scatter(db0, elg0, erg0, ee0, rows0)

        plsc.subcore_barrier()
        dbase = (cid * H + hh) * N
        for k in range(7):
            o = pl.ds(sid * STRIPE + k * CK, CK)
            pltpu.sync_copy(acc_sh.at[o], rows0)
            pltpu.sync_copy(rows0, acc_out.at[cid, hh, o])
            pltpu.sync_copy(den_sh.at[o], ee0)
            pltpu.sync_copy(ee0, den_out.at[pl.ds(dbase + sid * STRIPE
                                                  + k * CK, CK)])
        ot = pl.ds(sid * STRIPE + 7 * CK, 64)
        pltpu.sync_copy(acc_sh.at[ot], rows0.at[pl.ds(0, 64)])
        pltpu.sync_copy(rows0.at[pl.ds(0, 64)], acc_out.at[cid, hh, ot])
        pltpu.sync_copy(den_sh.at[ot], ee0.at[pl.ds(0, 64)])
        pltpu.sync_copy(ee0.at[pl.ds(0, 64)],
                        den_out.at[pl.ds(dbase + sid * STRIPE + 7 * CK, 64)])

        @pl.when(sid == NS - 1)
        def _():
            oz = pl.ds(NS * STRIPE, 16)
            pltpu.sync_copy(acc_sh.at[oz], rows1.at[pl.ds(0, 16)])
            pltpu.sync_copy(rows1.at[pl.ds(0, 16)], acc_out.at[cid, hh, oz])
            pltpu.sync_copy(den_sh.at[oz], ee1.at[pl.ds(0, 16)])
            pltpu.sync_copy(ee1.at[pl.ds(0, 16)],
                            den_out.at[pl.ds(dbase + NS * STRIPE, 16)])
        plsc.subcore_barrier()


@functools.lru_cache(maxsize=None)
def _gat_call():
    return pl.kernel(
        _gat_body,
        out_type=(jax.ShapeDtypeStruct((NC, H, N, D), F32),
                  jax.ShapeDtypeStruct((NC * H * N,), F32)),
        mesh=_mesh(),
        scratch_types=[pltpu.VMEM((EPW,), jnp.int32),
                       pltpu.VMEM((EPW,), jnp.int32),
                       pltpu.VMEM((CK,), jnp.int32),
                       pltpu.VMEM((CK,), jnp.int32),
                       pltpu.VMEM((CK,), jnp.int32),
                       pltpu.VMEM((CK,), jnp.int32),
                       pltpu.VMEM((CK,), jnp.int32),
                       pltpu.VMEM((CK,), jnp.int32),
                       pltpu.VMEM((CK,), F32),
                       pltpu.VMEM((CK,), F32),
                       pltpu.VMEM((CK,), F32),
                       pltpu.VMEM((CK,), F32),
                       pltpu.VMEM((CK,), F32),
                       pltpu.VMEM((CK,), F32),
                       pltpu.VMEM((CK, D), F32),
                       pltpu.VMEM((CK, D), F32),
                       pltpu.VMEM_SHARED((N, D), F32),
                       pltpu.VMEM_SHARED((N,), F32),
                       pltpu.SemaphoreType.DMA,
                       pltpu.SemaphoreType.DMA],
        interpret=_INTERP,
    )


def kernel(x, edge_index, edge_types, ggnn_W, ggnn_b, gru_Wih, gru_Whh,
           gru_bih, gru_bhh, gat_W, gat_attn_l, gat_attn_r, gat_b,
           conv_l1_W, conv_l1_b, avg_conv_l1_W, avg_conv_l1_b,
           conv_l2_W, conv_l2_b, avg_conv_l2_W, avg_conv_l2_b,
           conv_l1c_W, conv_l1c_b, avg_conv_l1c_W, avg_conv_l1c_b,
           conv_l2c_W, conv_l2c_b, avg_conv_l2c_W, avg_conv_l2c_b,
           mlp_h_W, mlp_h_b, mlp_W, mlp_b):
    src = edge_index[0]
    dst = edge_index[1]
    gixm = _eidx_call()(src.reshape(2500, 128),
                        edge_types.reshape(2500, 128))
    gix = gixm.reshape(E)

    bih = gru_bih.reshape(1, TD)
    bhh = gru_bhh.reshape(1, TD)
    gbm = ggnn_b.reshape(ET, D)

    h = x
    trans = _trans_call()(x, ggnn_W, gbm)
    for _step in range(STEPS):
        apart = _agg_call()(trans.reshape(ET * N, D), gix, dst)
        h, trans = _gru_call()(apart, h, gru_Wih, gru_Whh, bih, bhh,
                               ggnn_W, gbm)

    feat, lrt = _gatpre_call()(h, gat_W, gat_attn_l, gat_attn_r)
    lr = lrt.T.reshape(-1)
    acc, den = _gat_call()(feat.reshape(H * N, D), lr, src, dst)

    t210 = lambda w: jnp.transpose(w, (2, 1, 0))
    r1 = lambda v: v.reshape(1, -1)
    out = _readout_call()(
        acc.reshape(NC, H, B, NPG, D),
        den.reshape(NC, H, B, NPG).transpose(2, 0, 1, 3),
        x.reshape(B, NPG, D), gat_b.reshape(H, D),
        t210(conv_l1_W), r1(conv_l1_b), t210(avg_conv_l1_W), r1(avg_conv_l1_b),
        t210(conv_l2_W), r1(conv_l2_b), t210(avg_conv_l2_W), r1(avg_conv_l2_b),
        t210(conv_l1c_W), r1(conv_l1c_b),
        t210(avg_conv_l1c_W), r1(avg_conv_l1c_b),
        t210(conv_l2c_W), r1(conv_l2c_b),
        t210(avg_conv_l2c_W), r1(avg_conv_l2c_b),
        mlp_h_W.reshape(D, 1), mlp_h_b.reshape(1, 1),
        mlp_W.reshape(CD, 1), mlp_b.reshape(1, 1))
    return out[:, 0, 0]
